# baseline (device time: 12814 ns/iter reference)
import jax
import jax.numpy as jnp
from jax import lax
from jax.experimental import pallas as pl
from jax.experimental.pallas import tpu as pltpu

N_DEV = 4
N_TOK = 512
D_IN = 256
D_OUT = 512
N_EXP = 16
EXP_PER_DEV = 4
CHUNK = N_TOK // N_DEV


def kernel(x, router_W, route_idx, expert_W):
    del route_idx

    def body(x_ref, rw_ref, ew_ref, out_ref,
             sendbuf_ref, comm_ref, send_sems, recv_sems):
        my = lax.axis_index("i")

        barrier_sem = pltpu.get_barrier_semaphore()
        for o in range(1, N_DEV):
            pl.semaphore_signal(barrier_sem, inc=1,
                                device_id=(lax.rem(my + o, N_DEV),),
                                device_id_type=pl.DeviceIdType.MESH)

        rw = rw_ref[:, :]
        ewb = ew_ref[:, :, :].astype(jnp.bfloat16)
        eids = lax.broadcasted_iota(jnp.int32, (CHUNK, N_EXP), 1)

        def chunk_inputs(dest):
            xc = x_ref[pl.ds(dest * CHUNK, CHUNK), :]
            scores = jnp.dot(xc, rw, precision=lax.Precision.HIGHEST,
                             preferred_element_type=jnp.float32)
            m1 = jnp.max(scores, axis=1, keepdims=True)
            is_top1 = scores == m1
            m2 = jnp.max(jnp.where(is_top1, -jnp.inf, scores), axis=1,
                         keepdims=True)
            top2 = is_top1 | (scores == m2)
            p = jnp.exp(scores - m1)
            gated = jnp.where(top2, p, 0.0)
            gates = gated / jnp.sum(gated, axis=1, keepdims=True)
            xw = []
            for le in range(EXP_PER_DEV):
                ge = my * EXP_PER_DEV + le
                w = jnp.sum(jnp.where(eids == ge, gates, 0.0), axis=1,
                            keepdims=True)
                xw.append((xc * w).astype(jnp.bfloat16))
            return xw

        def half_partial(xw, h):
            cols = slice(h * (D_OUT // 2), (h + 1) * (D_OUT // 2))
            acc = jnp.zeros((CHUNK, D_OUT // 2), jnp.float32)
            for le in range(EXP_PER_DEV):
                acc = acc + jnp.dot(xw[le], ewb[le][:, cols],
                                    preferred_element_type=jnp.float32)
            return acc

        def chunk_partial(dest):
            xw = chunk_inputs(dest)
            return jnp.concatenate([half_partial(xw, 0), half_partial(xw, 1)],
                                   axis=1)

        HALF = D_OUT // 2
        send_rdmas = []
        for i, o in enumerate((2, 1, 3)):
            dest = lax.rem(my + o, N_DEV)
            slot = 3 - o
            xw = chunk_inputs(dest)
            for h in (0, 1):
                cols = slice(h * HALF, (h + 1) * HALF)
                sendbuf_ref[slot, :, cols] = half_partial(xw, h).astype(
                    jnp.bfloat16)
                if i == 0 and h == 0:
                    pl.semaphore_wait(barrier_sem, N_DEV - 1)
                rdma = pltpu.make_async_remote_copy(
                    src_ref=sendbuf_ref.at[slot, :, cols],
                    dst_ref=comm_ref.at[slot, :, cols],
                    send_sem=send_sems.at[slot, h],
                    recv_sem=recv_sems.at[slot, h],
                    device_id=(dest,),
                    device_id_type=pl.DeviceIdType.MESH,
                )
                rdma.start()
                send_rdmas.append(rdma)

        total = chunk_partial(my)

        for j in (1, 2, 0):
            for h in (0, 1):
                cols = slice(h * HALF, (h + 1) * HALF)
                recv = pltpu.make_async_remote_copy(
                    src_ref=sendbuf_ref.at[j, :, cols],
                    dst_ref=comm_ref.at[j, :, cols],
                    send_sem=send_sems.at[j, h],
                    recv_sem=recv_sems.at[j, h],
                    device_id=(my,),
                    device_id_type=pl.DeviceIdType.MESH,
                )
                recv.wait_recv()
            total = total + comm_ref[j, :, :].astype(jnp.float32)

        for rdma in send_rdmas:
            rdma.wait_send()

        out_ref[:, :] = total

    return pl.pallas_call(
        body,
        out_shape=jax.ShapeDtypeStruct((CHUNK, D_OUT), jnp.float32),
        in_specs=[pl.BlockSpec(memory_space=pltpu.VMEM)] * 3,
        out_specs=pl.BlockSpec(memory_space=pltpu.VMEM),
        scratch_shapes=[
            pltpu.VMEM((N_DEV - 1, CHUNK, D_OUT), jnp.bfloat16),
            pltpu.VMEM((N_DEV - 1, CHUNK, D_OUT), jnp.bfloat16),
            pltpu.SemaphoreType.DMA((N_DEV - 1, 2)),
            pltpu.SemaphoreType.DMA((N_DEV - 1, 2)),
        ],
        compiler_params=pltpu.CompilerParams(collective_id=0),
    )(x, router_W, expert_W)


# device time: 12635 ns/iter; 1.0142x vs baseline; 1.0142x over previous
import jax
import jax.numpy as jnp
from jax import lax
from jax.experimental import pallas as pl
from jax.experimental.pallas import tpu as pltpu

N_DEV = 4
N_TOK = 512
D_IN = 256
D_OUT = 512
N_EXP = 16
EXP_PER_DEV = 4
CHUNK = N_TOK // N_DEV


def kernel(x, router_W, route_idx, expert_W):
    del route_idx

    def body(x_ref, rw_ref, ew_ref, out_ref,
             sendbuf_ref, comm_ref, send_sems, recv_sems):
        my = lax.axis_index("i")

        barrier_sem = pltpu.get_barrier_semaphore()
        for o in range(1, N_DEV):
            pl.semaphore_signal(barrier_sem, inc=1,
                                device_id=(lax.rem(my + o, N_DEV),),
                                device_id_type=pl.DeviceIdType.MESH)

        rw = rw_ref[:, :]
        ewb = ew_ref[:, :, :].astype(jnp.bfloat16)
        eids = lax.broadcasted_iota(jnp.int32, (CHUNK, N_EXP), 1)

        def chunk_partial(dest):
            xc = x_ref[pl.ds(dest * CHUNK, CHUNK), :]
            scores = jnp.dot(xc, rw, precision=lax.Precision.HIGHEST,
                             preferred_element_type=jnp.float32)
            m1 = jnp.max(scores, axis=1, keepdims=True)
            is_top1 = scores == m1
            m2 = jnp.max(jnp.where(is_top1, -jnp.inf, scores), axis=1,
                         keepdims=True)
            top2 = is_top1 | (scores == m2)
            p = jnp.exp(scores - m1)
            gated = jnp.where(top2, p, 0.0)
            gates = gated / jnp.sum(gated, axis=1, keepdims=True)
            acc = jnp.zeros((CHUNK, D_OUT), jnp.float32)
            for le in range(EXP_PER_DEV):
                ge = my * EXP_PER_DEV + le
                w = jnp.sum(jnp.where(eids == ge, gates, 0.0), axis=1,
                            keepdims=True)
                acc = acc + jnp.dot((xc * w).astype(jnp.bfloat16), ewb[le],
                                    preferred_element_type=jnp.float32)
            return acc

        send_rdmas = []
        for i, o in enumerate((2, 1, 3)):
            dest = lax.rem(my + o, N_DEV)
            slot = 3 - o
            sendbuf_ref[slot, :, :] = chunk_partial(dest).astype(jnp.bfloat16)
            if i == 0:
                pl.semaphore_wait(barrier_sem, N_DEV - 1)
            rdma = pltpu.make_async_remote_copy(
                src_ref=sendbuf_ref.at[slot],
                dst_ref=comm_ref.at[slot],
                send_sem=send_sems.at[slot],
                recv_sem=recv_sems.at[slot],
                device_id=(dest,),
                device_id_type=pl.DeviceIdType.MESH,
            )
            rdma.start()
            send_rdmas.append(rdma)

        total = chunk_partial(my)

        for j in (1, 2, 0):
            recv = pltpu.make_async_remote_copy(
                src_ref=sendbuf_ref.at[j],
                dst_ref=comm_ref.at[j],
                send_sem=send_sems.at[j],
                recv_sem=recv_sems.at[j],
                device_id=(my,),
                device_id_type=pl.DeviceIdType.MESH,
            )
            recv.wait_recv()
            total = total + comm_ref[j, :, :].astype(jnp.float32)

        for rdma in send_rdmas:
            rdma.wait_send()

        out_ref[:, :] = total

    return pl.pallas_call(
        body,
        out_shape=jax.ShapeDtypeStruct((CHUNK, D_OUT), jnp.float32),
        in_specs=[pl.BlockSpec(memory_space=pltpu.VMEM)] * 3,
        out_specs=pl.BlockSpec(memory_space=pltpu.VMEM),
        scratch_shapes=[
            pltpu.VMEM((N_DEV - 1, CHUNK, D_OUT), jnp.bfloat16),
            pltpu.VMEM((N_DEV - 1, CHUNK, D_OUT), jnp.bfloat16),
            pltpu.SemaphoreType.DMA((N_DEV - 1,)),
            pltpu.SemaphoreType.DMA((N_DEV - 1,)),
        ],
        compiler_params=pltpu.CompilerParams(collective_id=0),
    )(x, router_W, expert_W)


# device time: 12597 ns/iter; 1.0172x vs baseline; 1.0030x over previous
import jax
import jax.numpy as jnp
from jax import lax
from jax.experimental import pallas as pl
from jax.experimental.pallas import tpu as pltpu

N_DEV = 4
N_TOK = 512
D_IN = 256
D_OUT = 512
N_EXP = 16
EXP_PER_DEV = 4
CHUNK = N_TOK // N_DEV


def kernel(x, router_W, route_idx, expert_W):
    del route_idx

    def body(x_ref, rw_ref, ew_ref, out_ref,
             sendbuf_ref, comm_ref, send_sems, recv_sems):
        my = lax.axis_index("i")

        barrier_sem = pltpu.get_barrier_semaphore()
        for o in range(1, N_DEV):
            pl.semaphore_signal(barrier_sem, inc=1,
                                device_id=(lax.rem(my + o, N_DEV),),
                                device_id_type=pl.DeviceIdType.MESH)

        rw = rw_ref[:, :]
        ewb = ew_ref[:, :, :].astype(jnp.bfloat16)
        eids = lax.broadcasted_iota(jnp.int32, (CHUNK, N_EXP), 1)

        def chunk_partial(dest):
            xc = x_ref[pl.ds(dest * CHUNK, CHUNK), :]
            scores = jnp.dot(xc, rw, precision=lax.Precision.HIGHEST,
                             preferred_element_type=jnp.float32)
            m1 = jnp.max(scores, axis=1, keepdims=True)
            is_top1 = scores == m1
            m2 = jnp.max(jnp.where(is_top1, -jnp.inf, scores), axis=1,
                         keepdims=True)
            top2 = is_top1 | (scores == m2)
            p = jnp.exp(scores - m1)
            gated = jnp.where(top2, p, 0.0)
            gates = gated / jnp.sum(gated, axis=1, keepdims=True)
            acc = jnp.zeros((CHUNK, D_OUT), jnp.float32)
            for le in range(EXP_PER_DEV):
                ge = my * EXP_PER_DEV + le
                w = jnp.sum(jnp.where(eids == ge, gates, 0.0), axis=1,
                            keepdims=True)
                acc = acc + jnp.dot((xc * w).astype(jnp.bfloat16), ewb[le],
                                    preferred_element_type=jnp.float32)
            return acc

        send_rdmas = []
        for i, o in enumerate((2, 1, 3)):
            dest = lax.rem(my + o, N_DEV)
            slot = 3 - o
            sendbuf_ref[slot, :, :] = chunk_partial(dest).astype(jnp.bfloat16)
            if i == 0:
                pl.semaphore_wait(barrier_sem, N_DEV - 1)
            rdma = pltpu.make_async_remote_copy(
                src_ref=sendbuf_ref.at[slot],
                dst_ref=comm_ref.at[slot],
                send_sem=send_sems.at[slot],
                recv_sem=recv_sems.at[slot],
                device_id=(dest,),
                device_id_type=pl.DeviceIdType.MESH,
            )
            rdma.start()
            send_rdmas.append(rdma)

        total = chunk_partial(my)

        for j in (1, 2, 0):
            recv = pltpu.make_async_remote_copy(
                src_ref=sendbuf_ref.at[j],
                dst_ref=comm_ref.at[j],
                send_sem=send_sems.at[j],
                recv_sem=recv_sems.at[j],
                device_id=(my,),
                device_id_type=pl.DeviceIdType.MESH,
            )
            recv.wait_recv()
            total = total + comm_ref[j, :, :].astype(jnp.float32)

        for rdma in send_rdmas:
            rdma.wait_send()

        out_ref[:, :] = total.astype(jnp.bfloat16)

    return pl.pallas_call(
        body,
        out_shape=jax.ShapeDtypeStruct((CHUNK, D_OUT), jnp.bfloat16),
        in_specs=[pl.BlockSpec(memory_space=pltpu.VMEM)] * 3,
        out_specs=pl.BlockSpec(memory_space=pltpu.VMEM),
        scratch_shapes=[
            pltpu.VMEM((N_DEV - 1, CHUNK, D_OUT), jnp.bfloat16),
            pltpu.VMEM((N_DEV - 1, CHUNK, D_OUT), jnp.bfloat16),
            pltpu.SemaphoreType.DMA((N_DEV - 1,)),
            pltpu.SemaphoreType.DMA((N_DEV - 1,)),
        ],
        compiler_params=pltpu.CompilerParams(collective_id=0),
    )(x, router_W, expert_W)
